# baseline (speedup 1.0000x reference)
import functools

import jax
import jax.numpy as jnp
from jax import lax
from jax.experimental import pallas as pl
from jax.experimental.pallas import tpu as pltpu
from jax.experimental.pallas import tpu_sc as plsc

_B, _IN, _OUT, _T, _C = 8192, 1024, 1024, 16, 10
_R = 1 << _C
_SPAD = 256
_LANES = 128

_NC, _NS, _L = 2, 16, 16
_NW = _NC * _NS
_NSPLIT = 2


def _idx_body(x_ref, s_ref, w_ref, o_ref):
    h = lax.dot(x_ref[...], s_ref[...],
                precision=lax.Precision.HIGHEST,
                preferred_element_type=jnp.float32)
    bits = (h > 0.0).astype(jnp.float32)
    idxf = lax.dot(bits, w_ref[...],
                   preferred_element_type=jnp.float32)
    off = lax.broadcasted_iota(jnp.int32, idxf.shape, 1) * _R
    o_ref[...] = idxf.astype(jnp.int32) + off


def _compute_idx(x, s_mat, w_mat):
    nb = x.shape[0]
    nblk = 4
    blk = nb // nblk
    return pl.pallas_call(
        _idx_body,
        grid=(nblk,),
        in_specs=[
            pl.BlockSpec((blk, _IN), lambda i: (i, 0)),
            pl.BlockSpec((_IN, _SPAD), lambda i: (0, 0)),
            pl.BlockSpec((_SPAD, _LANES), lambda i: (0, 0)),
        ],
        out_specs=pl.BlockSpec((blk, _LANES), lambda i: (i, 0)),
        out_shape=jax.ShapeDtypeStruct((nb, _LANES), jnp.int32),
    )(x, s_mat, w_mat)


@functools.cache
def _build_gather_sum(nb):
    bpw = nb // _NW
    ng = bpw
    @functools.partial(
        pl.kernel,
        mesh=plsc.VectorSubcoreMesh(core_axis_name="c", subcore_axis_name="s"),
        out_type=jax.ShapeDtypeStruct((nb, _OUT), jnp.float32),
        scratch_types=[
            pltpu.VMEM((bpw * _T,), jnp.int32),
            pltpu.VMEM((_T, _OUT), jnp.float32),
            pltpu.VMEM((_T, _OUT), jnp.float32),
            pltpu.VMEM((_T, _OUT), jnp.float32),
            pltpu.VMEM((_T, _OUT), jnp.float32),
            pltpu.VMEM((1, _OUT), jnp.float32),
            pltpu.VMEM((1, _OUT), jnp.float32),
            pltpu.SemaphoreType.DMA,
            pltpu.SemaphoreType.DMA,
            pltpu.SemaphoreType.DMA,
            pltpu.SemaphoreType.DMA,
            pltpu.SemaphoreType.DMA,
            pltpu.SemaphoreType.DMA,
        ],
    )
    def _gather_sum(tab_ref, idx_ref, y_ref, idx_v, r0, r1, r2, r3,
                    yb0, yb1, gs0, gs1, gs2, gs3, ys0, ys1):
        wid = lax.axis_index("s") * _NC + lax.axis_index("c")
        base = wid * bpw
        pltpu.sync_copy(idx_ref.at[pl.ds(base * _T, bpw * _T)], idx_v)

        rbufs = (r0, r1, r2, r3)
        gsems = (gs0, gs1, gs2, gs3)
        ybufs = (yb0, yb1)
        ysems = (ys0, ys1)

        def startg(g, buf, sem):
            pltpu.async_copy(tab_ref.at[idx_v.at[pl.ds(g * _T, _T)]],
                             buf, sem)

        startg(0, r0, gs0)
        startg(1, r1, gs1)
        startg(2, r2, gs2)

        def outer(h, carry):
            for b in range(4):
                g = 4 * h + b
                buf, sem = rbufs[b], gsems[b]
                p = b % 2
                ybuf, ysem = ybufs[p], ysems[p]
                pltpu.make_async_copy(
                    tab_ref.at[idx_v.at[pl.ds(0, _T)]], buf, sem).wait()

                @pl.when(g + 3 < ng)
                def _():
                    startg(g + 3, rbufs[(b + 3) % 4], gsems[(b + 3) % 4])

                @pl.when(g >= 2)
                def _():
                    pltpu.make_async_copy(
                        ybuf, y_ref.at[pl.ds(0, 1)], ysem).wait()

                @plsc.parallel_loop(0, _OUT, step=_L, unroll=8)
                def _(o):
                    vals = [buf[t, pl.ds(o, _L)] for t in range(_T)]
                    while len(vals) > 1:
                        nxt = [vals[i] + vals[i + 1]
                               for i in range(0, len(vals) - 1, 2)]
                        if len(vals) % 2:
                            nxt.append(vals[-1])
                        vals = nxt
                    ybuf[0, pl.ds(o, _L)] = vals[0]
                pltpu.async_copy(ybuf, y_ref.at[pl.ds(base + g, 1)], ysem)
            return carry

        lax.fori_loop(0, ng // 4, outer, 0)
        for p in range(2):
            pltpu.make_async_copy(
                ybufs[p], y_ref.at[pl.ds(0, 1)], ysems[p]).wait()

    return _gather_sum


def kernel(x, table, anchors_a, anchors_b, bit_powers):
    rows = jnp.arange(_IN, dtype=jnp.int32)[:, None]
    aa = jnp.full((_SPAD,), -1, jnp.int32).at[: _T * _C].set(
        anchors_a.reshape(-1))
    ab = jnp.full((_SPAD,), -1, jnp.int32).at[: _T * _C].set(
        anchors_b.reshape(-1))
    s_mat = ((rows == aa[None, :]).astype(jnp.float32)
             - (rows == ab[None, :]).astype(jnp.float32))
    cols = jnp.arange(_T * _C, dtype=jnp.int32)
    tt = jnp.repeat(jnp.arange(_T, dtype=jnp.int32), _C)
    w_mat = jnp.zeros((_SPAD, _LANES), jnp.float32)
    w_mat = w_mat.at[cols, tt].set(jnp.tile(bit_powers.astype(jnp.float32), _T))

    tab2 = table.reshape(_T * _R, _OUT)
    nb = _B // _NSPLIT
    ys = []
    for k in range(_NSPLIT):
        xk = lax.slice_in_dim(x, k * nb, (k + 1) * nb, axis=0)
        idx128 = _compute_idx(xk, s_mat, w_mat)
        flat_idx = idx128[:, :_T].reshape(-1)
        ys.append(_build_gather_sum(nb)(tab2, flat_idx))
    return jnp.concatenate(ys, axis=0)


# speedup vs baseline: 1.1772x; 1.1772x over previous
import functools

import jax
import jax.numpy as jnp
from jax import lax
from jax.experimental import pallas as pl
from jax.experimental.pallas import tpu as pltpu
from jax.experimental.pallas import tpu_sc as plsc

_B, _IN, _OUT, _T, _C = 8192, 1024, 1024, 16, 10
_R = 1 << _C
_SPAD = 256
_LANES = 128

_NC, _NS, _L = 2, 16, 16
_NW = _NC * _NS
_BPW = _B // _NW
_NG = _BPW


def _idx_body(x_ref, s_ref, w_ref, o_ref):
    h = lax.dot(x_ref[...], s_ref[...],
                precision=lax.Precision.HIGHEST,
                preferred_element_type=jnp.float32)
    bits = (h > 0.0).astype(jnp.float32)
    idxf = lax.dot(bits, w_ref[...],
                   preferred_element_type=jnp.float32)
    off = lax.broadcasted_iota(jnp.int32, idxf.shape, 1) * _R
    o_ref[...] = idxf.astype(jnp.int32) + off


def _compute_idx(x, s_mat, w_mat):
    nblk = 8
    blk = _B // nblk
    return pl.pallas_call(
        _idx_body,
        grid=(nblk,),
        in_specs=[
            pl.BlockSpec((blk, _IN), lambda i: (i, 0)),
            pl.BlockSpec((_IN, _SPAD), lambda i: (0, 0)),
            pl.BlockSpec((_SPAD, _LANES), lambda i: (0, 0)),
        ],
        out_specs=pl.BlockSpec((blk, _LANES), lambda i: (i, 0)),
        out_shape=jax.ShapeDtypeStruct((_B, _LANES), jnp.int32),
    )(x, s_mat, w_mat)


@functools.cache
def _build_gather_sum():
    @functools.partial(
        pl.kernel,
        mesh=plsc.VectorSubcoreMesh(core_axis_name="c", subcore_axis_name="s"),
        out_type=jax.ShapeDtypeStruct((_B, _OUT), jnp.float32),
        scratch_types=[
            pltpu.VMEM((_BPW * _T,), jnp.int32),
            pltpu.VMEM((_T, _OUT), jnp.float32),
            pltpu.VMEM((_T, _OUT), jnp.float32),
            pltpu.VMEM((_T, _OUT), jnp.float32),
            pltpu.VMEM((_T, _OUT), jnp.float32),
            pltpu.VMEM((1, _OUT), jnp.float32),
            pltpu.VMEM((1, _OUT), jnp.float32),
            pltpu.SemaphoreType.DMA,
            pltpu.SemaphoreType.DMA,
            pltpu.SemaphoreType.DMA,
            pltpu.SemaphoreType.DMA,
            pltpu.SemaphoreType.DMA,
            pltpu.SemaphoreType.DMA,
        ],
    )
    def _gather_sum(tab_ref, idx_ref, y_ref, idx_v, r0, r1, r2, r3,
                    yb0, yb1, gs0, gs1, gs2, gs3, ys0, ys1):
        wid = lax.axis_index("s") * _NC + lax.axis_index("c")
        base = wid * _BPW
        pltpu.sync_copy(idx_ref.at[pl.ds(base * _T, _BPW * _T)], idx_v)

        rbufs = (r0, r1, r2, r3)
        gsems = (gs0, gs1, gs2, gs3)
        ybufs = (yb0, yb1)
        ysems = (ys0, ys1)

        def startg(g, buf, sem):
            pltpu.async_copy(tab_ref.at[idx_v.at[pl.ds(g * _T, _T)]],
                             buf, sem)

        startg(0, r0, gs0)
        startg(1, r1, gs1)
        startg(2, r2, gs2)

        def outer(h, carry):
            for b in range(4):
                g = 4 * h + b
                buf, sem = rbufs[b], gsems[b]
                p = b % 2
                ybuf, ysem = ybufs[p], ysems[p]
                pltpu.make_async_copy(
                    tab_ref.at[idx_v.at[pl.ds(0, _T)]], buf, sem).wait()

                @pl.when(g + 3 < _NG)
                def _():
                    startg(g + 3, rbufs[(b + 3) % 4], gsems[(b + 3) % 4])

                @pl.when(g >= 2)
                def _():
                    pltpu.make_async_copy(
                        ybuf, y_ref.at[pl.ds(0, 1)], ysem).wait()

                @plsc.parallel_loop(0, _OUT, step=_L, unroll=8)
                def _(o):
                    vals = [buf[t, pl.ds(o, _L)] for t in range(_T)]
                    while len(vals) > 1:
                        nxt = [vals[i] + vals[i + 1]
                               for i in range(0, len(vals) - 1, 2)]
                        if len(vals) % 2:
                            nxt.append(vals[-1])
                        vals = nxt
                    ybuf[0, pl.ds(o, _L)] = vals[0]
                pltpu.async_copy(ybuf, y_ref.at[pl.ds(base + g, 1)], ysem)
            return carry

        lax.fori_loop(0, _NG // 4, outer, 0)
        for p in range(2):
            pltpu.make_async_copy(
                ybufs[p], y_ref.at[pl.ds(0, 1)], ysems[p]).wait()

    return _gather_sum


def kernel(x, table, anchors_a, anchors_b, bit_powers):
    rows = jnp.arange(_IN, dtype=jnp.int32)[:, None]
    aa = jnp.full((_SPAD,), -1, jnp.int32).at[: _T * _C].set(
        anchors_a.reshape(-1))
    ab = jnp.full((_SPAD,), -1, jnp.int32).at[: _T * _C].set(
        anchors_b.reshape(-1))
    s_mat = ((rows == aa[None, :]).astype(jnp.float32)
             - (rows == ab[None, :]).astype(jnp.float32))
    cols = jnp.arange(_T * _C, dtype=jnp.int32)
    tt = jnp.repeat(jnp.arange(_T, dtype=jnp.int32), _C)
    w_mat = jnp.zeros((_SPAD, _LANES), jnp.float32)
    w_mat = w_mat.at[cols, tt].set(jnp.tile(bit_powers.astype(jnp.float32), _T))

    idx128 = _compute_idx(x, s_mat, w_mat)
    flat_idx = idx128[:, :_T].reshape(-1)
    y = _build_gather_sum()(table.reshape(_T * _R, _OUT), flat_idx)
    return y
